# baseline (speedup 1.0000x reference)
import jax
import jax.numpy as jnp
from jax import lax
from jax.experimental import pallas as pl
from jax.experimental.pallas import tpu as pltpu


_BLK = 1024
_NBUF = 4
_NSPLIT = 4


def _gate_kernel(x_hbm, wt_ref, out_ref, buf, sems):
    n_chunks = x_hbm.shape[0] // _BLK
    sub = _BLK // _NSPLIT
    wt = wt_ref[...]

    def mk_copy(chunk, slot, j):
        return pltpu.make_async_copy(
            x_hbm.at[pl.ds(chunk * _BLK + j * sub, sub), :],
            buf.at[slot, pl.ds(j * sub, sub), :],
            sems.at[slot, j],
        )

    def start_copy(chunk, slot):
        for j in range(_NSPLIT):
            mk_copy(chunk, slot, j).start()

    def wait_copy(chunk, slot):
        for j in range(_NSPLIT):
            mk_copy(chunk, slot, j).wait()

    for b in range(_NBUF - 1):
        start_copy(b, b)

    def step(i, carry):
        slot = lax.rem(i, _NBUF)
        nxt = i + _NBUF - 1

        @pl.when(nxt < n_chunks)
        def _():
            start_copy(nxt, lax.rem(nxt, _NBUF))

        wait_copy(i, slot)
        out_ref[pl.ds(i * _BLK, _BLK), :] = jnp.dot(
            buf[slot], wt, preferred_element_type=jnp.float32)
        return carry

    lax.fori_loop(0, n_chunks, step, 0)


def kernel(hidden_states, weight):
    bsz, seq_len, h = hidden_states.shape
    n_exp = weight.shape[0]
    rows = bsz * seq_len
    x = hidden_states.reshape(rows, h)
    wt = weight.T

    out = pl.pallas_call(
        _gate_kernel,
        in_specs=[
            pl.BlockSpec(memory_space=pltpu.HBM),
            pl.BlockSpec(memory_space=pltpu.VMEM),
        ],
        out_specs=pl.BlockSpec(memory_space=pltpu.VMEM),
        out_shape=jax.ShapeDtypeStruct((rows, n_exp), jnp.float32),
        scratch_shapes=[
            pltpu.VMEM((_NBUF, _BLK, h), jnp.float32),
            pltpu.SemaphoreType.DMA((_NBUF, _NSPLIT)),
        ],
        compiler_params=pltpu.CompilerParams(
            vmem_limit_bytes=100 * 1024 * 1024,
        ),
    )(x, wt)
    return out


# speedup vs baseline: 1.0416x; 1.0416x over previous
import jax
import jax.numpy as jnp
from jax.experimental import pallas as pl
from jax.experimental.pallas import tpu as pltpu


_NSTREAM = 4
_ROWS_PER_BLOCK = 512


def _gate_kernel(*refs):
    x_refs = refs[:_NSTREAM]
    wt_ref = refs[_NSTREAM]
    out_ref = refs[_NSTREAM + 1]
    wt = wt_ref[...]
    for s in range(_NSTREAM):
        out_ref[s] = jnp.dot(x_refs[s][0], wt,
                             preferred_element_type=jnp.float32)


def kernel(hidden_states, weight):
    bsz, seq_len, h = hidden_states.shape
    n_exp = weight.shape[0]
    rows = bsz * seq_len
    x = hidden_states.reshape(_NSTREAM, rows // _NSTREAM, h)
    wt = weight.T

    per_stream = rows // _NSTREAM
    grid = (per_stream // _ROWS_PER_BLOCK,)
    out = pl.pallas_call(
        _gate_kernel,
        grid=grid,
        in_specs=[
            pl.BlockSpec((1, _ROWS_PER_BLOCK, h),
                         lambda i, s=s: (s, i, 0))
            for s in range(_NSTREAM)
        ] + [pl.BlockSpec((h, n_exp), lambda i: (0, 0))],
        out_specs=pl.BlockSpec((_NSTREAM, _ROWS_PER_BLOCK, n_exp),
                               lambda i: (0, i, 0)),
        out_shape=jax.ShapeDtypeStruct((_NSTREAM, per_stream, n_exp),
                                       jnp.float32),
        compiler_params=pltpu.CompilerParams(
            dimension_semantics=(pltpu.PARALLEL,),
        ),
    )(*([x] * _NSTREAM + [wt]))
    return out.reshape(rows, n_exp)
